# Initial kernel scaffold; baseline (speedup 1.0000x reference)
import jax
import jax.numpy as jnp
from jax.experimental import pallas as pl


def kernel(input, boxes):
    raise NotImplementedError("write your pallas kernel here")


# speedup vs baseline: 1.9850x; 1.9850x over previous
import functools

import jax
import jax.numpy as jnp
from jax import lax
from jax.experimental import pallas as pl
from jax.experimental.pallas import tpu as pltpu
from jax.experimental.pallas import tpu_sc as plsc

_P = 7
_S = 2
_SCALE = 0.25
_L = 16
_NC = 2
_NS = 16
_NW = _NC * _NS


def _take(vec, i):
    idx = jnp.full((_L, 1), i, jnp.int32)
    dnums = lax.GatherDimensionNumbers(
        offset_dims=(), collapsed_slice_dims=(0,), start_index_map=(0,))
    return lax.gather(vec, idx, dnums, (1,),
                      mode=lax.GatherScatterMode.PROMISE_IN_BOUNDS)


@functools.lru_cache(maxsize=None)
def _make_roi_kernel(N, C, H, W, M):
    BPW = M // _NW
    PP = _P * _P
    mesh = plsc.VectorSubcoreMesh(core_axis_name="c", subcore_axis_name="s")

    @functools.partial(
        pl.kernel,
        mesh=mesh,
        out_type=jax.ShapeDtypeStruct((M, PP, C), jnp.float32),
        scratch_types=[
            pltpu.VMEM((5 * BPW,), jnp.float32),
            pltpu.VMEM((8 * _L,), jnp.int32),
            pltpu.VMEM((8 * _L, C), jnp.float32),
            pltpu.VMEM((PP, C), jnp.float32),
            pltpu.SemaphoreType.DMA,
        ],
    )
    def roi_sc(table_hbm, boxes_hbm, out_hbm, boxes_v, idx_v, g_v, o_v, sem):
        wid = lax.axis_index("s") * _NC + lax.axis_index("c")
        for f in range(5):
            pltpu.sync_copy(boxes_hbm.at[pl.ds(f * M + wid * BPW, BPW)],
                            boxes_v.at[pl.ds(f * BPW, BPW)])

        def box_body(bi, carry):
            chunk = (bi // _L) * _L
            lane = bi - chunk

            def field(f):
                return _take(boxes_v[pl.ds(f * BPW + chunk, _L)], lane)

            b_f = field(0)
            x1 = field(1) * _SCALE - 0.5
            y1 = field(2) * _SCALE - 0.5
            x2 = field(3) * _SCALE - 0.5
            y2 = field(4) * _SCALE - 0.5
            binw = (x2 - x1) * (1.0 / _P)
            binh = (y2 - y1) * (1.0 / _P)
            t = lax.iota(jnp.int32, _L)
            grid = (t >> 1).astype(jnp.float32) + ((t & 1).astype(jnp.float32) + 0.5) * (1.0 / _S)
            xs = x1 + grid * binw
            ys = y1 + grid * binh

            def prep(v, L):
                valid = (v >= -1.0) & (v <= float(L))
                v = jnp.maximum(v, 0.0)
                low = jnp.minimum(v.astype(jnp.int32), L - 1)
                high = jnp.minimum(low + 1, L - 1)
                frac = jnp.where(low >= L - 1, 0.0, v - low.astype(jnp.float32))
                wlo = jnp.where(valid, (1.0 - frac) * (1.0 / _S), 0.0)
                whi = jnp.where(valid, frac * (1.0 / _S), 0.0)
                return low, high, wlo, whi

            yl, yh, wloy, whiy = prep(ys, H)
            xl, xh, wlox, whix = prep(xs, W)
            bbase = b_f.astype(jnp.int32) * (H * W)

            def p_body(p, carry2):
                for i_off in range(_S):
                    i = _S * p + i_off
                    for yc in range(2):
                        yv = _take(yl if yc == 0 else yh, i)
                        row_base = bbase + yv * W
                        for xc in range(2):
                            g = i_off * 4 + yc * 2 + xc
                            idx_v[pl.ds(g * _L, _L)] = row_base + (xl if xc == 0 else xh)
                pltpu.async_copy(table_hbm.at[idx_v], g_v, sem).wait()

                wy = []
                for i_off in range(_S):
                    i = _S * p + i_off
                    wy.append((_take(wloy, i), _take(whiy, i)))

                def q_body(q, carry3):
                    terms = []
                    for j_off in range(_S):
                        j = _S * q + j_off
                        wx = (_take(wlox, j), _take(whix, j))
                        for i_off in range(_S):
                            for yc in range(2):
                                for xc in range(2):
                                    w = wy[i_off][yc] * wx[xc]
                                    row = (i_off * 4 + yc * 2 + xc) * _L + j
                                    terms.append((w, row))
                    for k in range(C // _L):
                        acc = None
                        for (w, row) in terms:
                            v = g_v[row, pl.ds(k * _L, _L)]
                            acc = w * v if acc is None else acc + w * v
                        o_v[p * _P + q, pl.ds(k * _L, _L)] = acc
                    return carry3

                lax.fori_loop(0, _P, q_body, None)
                return carry2

            lax.fori_loop(0, _P, p_body, None)
            pltpu.sync_copy(o_v, out_hbm.at[wid * BPW + bi])
            return carry

        lax.fori_loop(0, BPW, box_body, None)

    return roi_sc


def kernel(input, boxes):
    N, C, H, W = input.shape
    M = boxes.shape[0]
    table = jnp.transpose(input, (0, 2, 3, 1)).reshape(N * H * W, C)
    roi_sc = _make_roi_kernel(N, C, H, W, M)
    out = roi_sc(table, jnp.transpose(boxes).reshape(-1))
    return jnp.transpose(out, (0, 2, 1)).reshape(M, C, _P, _P)


# speedup vs baseline: 3.4606x; 1.7434x over previous
import functools

import jax
import jax.numpy as jnp
from jax import lax
from jax.experimental import pallas as pl
from jax.experimental.pallas import tpu as pltpu
from jax.experimental.pallas import tpu_sc as plsc

_P = 7
_S = 2
_SCALE = 0.25
_L = 16
_NC = 2
_NS = 16
_NW = _NC * _NS


def _take(vec, i):
    idx = jnp.full((_L, 1), i, jnp.int32)
    dnums = lax.GatherDimensionNumbers(
        offset_dims=(), collapsed_slice_dims=(0,), start_index_map=(0,))
    return lax.gather(vec, idx, dnums, (1,),
                      mode=lax.GatherScatterMode.PROMISE_IN_BOUNDS)


@functools.lru_cache(maxsize=None)
def _make_roi_kernel(N, C, H, W, M):
    BPW = M // _NW
    PP = _P * _P
    mesh = plsc.VectorSubcoreMesh(core_axis_name="c", subcore_axis_name="s")

    @functools.partial(
        pl.kernel,
        mesh=mesh,
        out_type=jax.ShapeDtypeStruct((M, PP, C), jnp.float32),
        scratch_types=[
            pltpu.VMEM((5 * BPW,), jnp.float32),
            pltpu.VMEM((2, 8 * _L), jnp.int32),
            pltpu.VMEM((2, 8 * _L, C), jnp.float32),
            pltpu.VMEM((PP, C), jnp.float32),
            pltpu.SemaphoreType.DMA,
        ],
    )
    def roi_sc(table_hbm, boxes_hbm, out_hbm, boxes_v, idx_v, g_v, o_v, sem):
        wid = lax.axis_index("s") * _NC + lax.axis_index("c")
        for f in range(5):
            pltpu.sync_copy(boxes_hbm.at[pl.ds(f * M + wid * BPW, BPW)],
                            boxes_v.at[pl.ds(f * BPW, BPW)])

        def box_body(bi, carry):
            chunk = (bi // _L) * _L
            lane = bi - chunk

            def field(f):
                return _take(boxes_v[pl.ds(f * BPW + chunk, _L)], lane)

            b_f = field(0)
            x1 = field(1) * _SCALE - 0.5
            y1 = field(2) * _SCALE - 0.5
            x2 = field(3) * _SCALE - 0.5
            y2 = field(4) * _SCALE - 0.5
            binw = (x2 - x1) * (1.0 / _P)
            binh = (y2 - y1) * (1.0 / _P)
            t = lax.iota(jnp.int32, _L)
            grid = (t >> 1).astype(jnp.float32) + ((t & 1).astype(jnp.float32) + 0.5) * (1.0 / _S)
            xs = x1 + grid * binw
            ys = y1 + grid * binh

            def prep(v, L):
                valid = (v >= -1.0) & (v <= float(L))
                v = jnp.maximum(v, 0.0)
                low = jnp.minimum(v.astype(jnp.int32), L - 1)
                high = jnp.minimum(low + 1, L - 1)
                frac = jnp.where(low >= L - 1, 0.0, v - low.astype(jnp.float32))
                wlo = jnp.where(valid, (1.0 - frac) * (1.0 / _S), 0.0)
                whi = jnp.where(valid, frac * (1.0 / _S), 0.0)
                return low, high, wlo, whi

            yl, yh, wloy, whiy = prep(ys, H)
            xl, xh, wlox, whix = prep(xs, W)
            bbase = b_f.astype(jnp.int32) * (H * W)

            def start_gather(p):
                par = p & 1
                for i_off in range(_S):
                    i = _S * p + i_off
                    for yc in range(2):
                        yv = _take(yl if yc == 0 else yh, i)
                        row_base = bbase + yv * W
                        for xc in range(2):
                            g = i_off * 4 + yc * 2 + xc
                            idx_v[par, pl.ds(g * _L, _L)] = row_base + (xl if xc == 0 else xh)
                pltpu.async_copy(table_hbm.at[idx_v.at[par]], g_v.at[par], sem)

            start_gather(0)

            def p_body(p, carry2):
                par = p & 1

                @pl.when(p < _P - 1)
                def _():
                    start_gather(p + 1)

                pltpu.make_async_copy(table_hbm.at[idx_v.at[par]], g_v.at[par], sem).wait()

                wy = []
                for i_off in range(_S):
                    i = _S * p + i_off
                    wy.append((_take(wloy, i), _take(whiy, i)))

                def q_body(q, carry3):
                    terms = []
                    for j_off in range(_S):
                        j = _S * q + j_off
                        wx = (_take(wlox, j), _take(whix, j))
                        for i_off in range(_S):
                            for yc in range(2):
                                for xc in range(2):
                                    w = wy[i_off][yc] * wx[xc]
                                    row = (i_off * 4 + yc * 2 + xc) * _L + j
                                    terms.append((w, row))
                    for k in range(C // _L):
                        prods = [w * g_v[par, row, pl.ds(k * _L, _L)]
                                 for (w, row) in terms]
                        while len(prods) > 1:
                            prods = [prods[z] + prods[z + 1]
                                     for z in range(0, len(prods), 2)]
                        o_v[p * _P + q, pl.ds(k * _L, _L)] = prods[0]
                    return carry3

                lax.fori_loop(0, _P, q_body, None)
                return carry2

            lax.fori_loop(0, _P, p_body, None)
            pltpu.sync_copy(o_v, out_hbm.at[wid * BPW + bi])
            return carry

        lax.fori_loop(0, BPW, box_body, None)

    return roi_sc


def kernel(input, boxes):
    N, C, H, W = input.shape
    M = boxes.shape[0]
    table = jnp.transpose(input, (0, 2, 3, 1)).reshape(N * H * W, C)
    roi_sc = _make_roi_kernel(N, C, H, W, M)
    out = roi_sc(table, jnp.transpose(boxes).reshape(-1))
    return jnp.transpose(out, (0, 2, 1)).reshape(M, C, _P, _P)
